# baseline (speedup 1.0000x reference)
import functools

import jax
import jax.numpy as jnp
from jax import lax
from jax.experimental import pallas as pl
from jax.experimental.pallas import tpu as pltpu
from jax.experimental.pallas import tpu_sc as plsc

B = 8
N = 2048
L = 16
NKV = N // L
QBLK = 4
UNROLL = 4

KSC = 0

TILE_N = 2048
NB = N // TILE_N


_CHUNKS = 16 // KSC if KSC else 1
_QLEN = N // _CHUNKS if KSC else N


def _round_bf16(v):
    u = plsc.bitcast(v, jnp.uint32)
    lsb = (u >> jnp.uint32(16)) & jnp.uint32(1)
    r = (u + jnp.uint32(0x7FFF) + lsb) & jnp.uint32(0xFFFF0000)
    return plsc.bitcast(r, jnp.float32)


def _sc_tile_body(pts_hbm, out_hbm, qx, qy, qz, q2, kx, ky, kz, k2, res):
    c = lax.axis_index("c")
    s = lax.axis_index("s")
    w = s * 2 + c
    d = w // 16
    r = w % 16
    b = r // _CHUNKS
    h = r % _CHUNKS
    q0 = h * _QLEN
    dk = 1 - d

    pltpu.sync_copy(pts_hbm.at[d, 0, b, pl.ds(q0, _QLEN)], qx)
    pltpu.sync_copy(pts_hbm.at[d, 1, b, pl.ds(q0, _QLEN)], qy)
    pltpu.sync_copy(pts_hbm.at[d, 2, b, pl.ds(q0, _QLEN)], qz)
    pltpu.sync_copy(pts_hbm.at[dk, 0, b], kx)
    pltpu.sync_copy(pts_hbm.at[dk, 1, b], ky)
    pltpu.sync_copy(pts_hbm.at[dk, 2, b], kz)

    def k2_body(j, carry):
        off = j * L
        xs = kx[pl.ds(off, L)]
        ys = ky[pl.ds(off, L)]
        zs = kz[pl.ds(off, L)]
        k2[pl.ds(off, L)] = xs * xs + ys * ys + zs * zs
        kx[pl.ds(off, L)] = _round_bf16(xs)
        ky[pl.ds(off, L)] = _round_bf16(ys)
        kz[pl.ds(off, L)] = _round_bf16(zs)
        return carry

    lax.fori_loop(0, NKV, k2_body, 0)

    def q2_body(j, carry):
        off = j * L
        xs = qx[pl.ds(off, L)]
        ys = qy[pl.ds(off, L)]
        zs = qz[pl.ds(off, L)]
        q2[pl.ds(off, L)] = xs * xs + ys * ys + zs * zs
        qx[pl.ds(off, L)] = _round_bf16(xs)
        qy[pl.ds(off, L)] = _round_bf16(ys)
        qz[pl.ds(off, L)] = _round_bf16(zs)
        return carry

    lax.fori_loop(0, _QLEN // L, q2_body, 0)

    inf = jnp.full((L,), jnp.inf, jnp.float32)

    def qblk_body(i, carry):
        qbase = i * QBLK
        av = []
        bv = []
        cv = []
        q2v = []
        for u in range(QBLK):
            idx = jnp.full((L,), qbase + u, jnp.int32)
            av.append(-2.0 * plsc.load_gather(qx, [idx]))
            bv.append(-2.0 * plsc.load_gather(qy, [idx]))
            cv.append(-2.0 * plsc.load_gather(qz, [idx]))
            q2v.append(plsc.load_gather(q2, [idx]))

        def k_body(jj, accs):
            accs = list(accs)
            for t in range(UNROLL):
                off = (jj * UNROLL + t) * L
                xk = kx[pl.ds(off, L)]
                yk = ky[pl.ds(off, L)]
                zk = kz[pl.ds(off, L)]
                kk = k2[pl.ds(off, L)]
                for u in range(QBLK):
                    dv = kk + av[u] * xk + bv[u] * yk + cv[u] * zk
                    accs[u] = jnp.minimum(accs[u], dv)
            return tuple(accs)

        accs = lax.fori_loop(0, NKV // UNROLL, k_body,
                             (inf,) * QBLK)
        lane0 = lax.iota(jnp.int32, L) == 0
        for u in range(QBLK):
            m = jnp.min(accs[u] + q2v[u])
            idx = jnp.full((L,), qbase + u, jnp.int32)
            plsc.store_scatter(res, [idx], jnp.full((L,), m), mask=lane0)
        return carry

    lax.fori_loop(0, _QLEN // QBLK, qblk_body, 0)

    pltpu.sync_copy(res, out_hbm.at[d, b, pl.ds(q0, _QLEN)])


def _make_sc_kernel():
    mesh = plsc.VectorSubcoreMesh(core_axis_name="c", subcore_axis_name="s")
    return pl.kernel(
        _sc_tile_body,
        out_type=jax.ShapeDtypeStruct((2, KSC, N), jnp.float32),
        mesh=mesh,
        scratch_types=[
            pltpu.VMEM((_QLEN,), jnp.float32),
            pltpu.VMEM((_QLEN,), jnp.float32),
            pltpu.VMEM((_QLEN,), jnp.float32),
            pltpu.VMEM((_QLEN,), jnp.float32),
            pltpu.VMEM((N,), jnp.float32),
            pltpu.VMEM((N,), jnp.float32),
            pltpu.VMEM((N,), jnp.float32),
            pltpu.VMEM((N,), jnp.float32),
            pltpu.VMEM((_QLEN,), jnp.float32),
        ],
        compiler_params=pltpu.CompilerParams(needs_layout_passes=False),
    )


def _tc_batch(xb, yt, d1_ref, d2_ref, b):
    n1 = jnp.sum(xb * xb, axis=1, keepdims=True)
    n2 = jnp.sum(yt * yt, axis=0, keepdims=True)
    hi = n2.astype(jnp.bfloat16)
    r1 = n2 - hi.astype(jnp.float32)
    mid = r1.astype(jnp.bfloat16)
    lo = (r1 - mid.astype(jnp.float32)).astype(jnp.bfloat16)
    a = (-2.0 * xb).astype(jnp.bfloat16)
    ones = jnp.ones((N, 3), jnp.bfloat16)
    lhs = jnp.concatenate([a, ones], axis=1)
    rhs = jnp.concatenate(
        [yt.astype(jnp.bfloat16), hi, mid, lo], axis=0)
    CW = N // 4
    d1acc = None
    for c in range(N // CW):
        t = lax.dot_general(lhs, rhs[:, c * CW:(c + 1) * CW],
                            (((1,), (0,)), ((), ())),
                            preferred_element_type=jnp.float32)
        m = jnp.min(t, axis=1)
        d1acc = m if d1acc is None else jnp.minimum(d1acc, m)
        d2_ref[b, 0, c * CW:(c + 1) * CW] = jnp.min(t + n1, axis=0)
    d1_ref[b, 0, :] = d1acc + n1[:, 0]


def _tc_body(x1_ref, p2t_ref, d1_ref, d2_ref):
    nb = x1_ref.shape[0]

    def body(b, carry):
        _tc_batch(x1_ref[b], p2t_ref[b], d1_ref, d2_ref, b)
        return carry

    lax.fori_loop(0, nb, body, 0)


def _tc_pair(x1, x2t, nb_batches):
    return pl.pallas_call(
        _tc_body,
        out_shape=[
            jax.ShapeDtypeStruct((nb_batches, 1, N), jnp.float32),
            jax.ShapeDtypeStruct((nb_batches, 1, N), jnp.float32),
        ],
    )(x1, x2t)


def kernel(xyz1, xyz2):
    outs1 = []
    outs2 = []
    if KSC:
        pts = jnp.stack([
            jnp.moveaxis(xyz1[:KSC], -1, 0),
            jnp.moveaxis(xyz2[:KSC], -1, 0),
        ])
        sc_out = _make_sc_kernel()(pts)
        outs1.append(sc_out[0])
        outs2.append(sc_out[1])
    if KSC < B:
        ntc = B - KSC
        x1 = xyz1[KSC:]
        x2 = xyz2[KSC:]
        x2t = jnp.moveaxis(x2, -1, 1)
        d1, d2 = _tc_pair(x1, x2t, ntc)
        outs1.append(d1.reshape(ntc, N))
        outs2.append(d2.reshape(ntc, N))
    dist1 = outs1[0] if len(outs1) == 1 else jnp.concatenate(outs1, axis=0)
    dist2 = outs2[0] if len(outs2) == 1 else jnp.concatenate(outs2, axis=0)
    return dist1, dist2


# speedup vs baseline: 1.0019x; 1.0019x over previous
import functools

import jax
import jax.numpy as jnp
from jax import lax
from jax.experimental import pallas as pl
from jax.experimental.pallas import tpu as pltpu
from jax.experimental.pallas import tpu_sc as plsc

B = 8
N = 2048
L = 16
NKV = N // L
QBLK = 4
UNROLL = 4

KSC = 0

TILE_N = 2048
NB = N // TILE_N


_CHUNKS = 16 // KSC if KSC else 1
_QLEN = N // _CHUNKS if KSC else N


def _round_bf16(v):
    u = plsc.bitcast(v, jnp.uint32)
    lsb = (u >> jnp.uint32(16)) & jnp.uint32(1)
    r = (u + jnp.uint32(0x7FFF) + lsb) & jnp.uint32(0xFFFF0000)
    return plsc.bitcast(r, jnp.float32)


def _sc_tile_body(pts_hbm, out_hbm, qx, qy, qz, q2, kx, ky, kz, k2, res):
    c = lax.axis_index("c")
    s = lax.axis_index("s")
    w = s * 2 + c
    d = w // 16
    r = w % 16
    b = r // _CHUNKS
    h = r % _CHUNKS
    q0 = h * _QLEN
    dk = 1 - d

    pltpu.sync_copy(pts_hbm.at[d, 0, b, pl.ds(q0, _QLEN)], qx)
    pltpu.sync_copy(pts_hbm.at[d, 1, b, pl.ds(q0, _QLEN)], qy)
    pltpu.sync_copy(pts_hbm.at[d, 2, b, pl.ds(q0, _QLEN)], qz)
    pltpu.sync_copy(pts_hbm.at[dk, 0, b], kx)
    pltpu.sync_copy(pts_hbm.at[dk, 1, b], ky)
    pltpu.sync_copy(pts_hbm.at[dk, 2, b], kz)

    def k2_body(j, carry):
        off = j * L
        xs = kx[pl.ds(off, L)]
        ys = ky[pl.ds(off, L)]
        zs = kz[pl.ds(off, L)]
        k2[pl.ds(off, L)] = xs * xs + ys * ys + zs * zs
        kx[pl.ds(off, L)] = _round_bf16(xs)
        ky[pl.ds(off, L)] = _round_bf16(ys)
        kz[pl.ds(off, L)] = _round_bf16(zs)
        return carry

    lax.fori_loop(0, NKV, k2_body, 0)

    def q2_body(j, carry):
        off = j * L
        xs = qx[pl.ds(off, L)]
        ys = qy[pl.ds(off, L)]
        zs = qz[pl.ds(off, L)]
        q2[pl.ds(off, L)] = xs * xs + ys * ys + zs * zs
        qx[pl.ds(off, L)] = _round_bf16(xs)
        qy[pl.ds(off, L)] = _round_bf16(ys)
        qz[pl.ds(off, L)] = _round_bf16(zs)
        return carry

    lax.fori_loop(0, _QLEN // L, q2_body, 0)

    inf = jnp.full((L,), jnp.inf, jnp.float32)

    def qblk_body(i, carry):
        qbase = i * QBLK
        av = []
        bv = []
        cv = []
        q2v = []
        for u in range(QBLK):
            idx = jnp.full((L,), qbase + u, jnp.int32)
            av.append(-2.0 * plsc.load_gather(qx, [idx]))
            bv.append(-2.0 * plsc.load_gather(qy, [idx]))
            cv.append(-2.0 * plsc.load_gather(qz, [idx]))
            q2v.append(plsc.load_gather(q2, [idx]))

        def k_body(jj, accs):
            accs = list(accs)
            for t in range(UNROLL):
                off = (jj * UNROLL + t) * L
                xk = kx[pl.ds(off, L)]
                yk = ky[pl.ds(off, L)]
                zk = kz[pl.ds(off, L)]
                kk = k2[pl.ds(off, L)]
                for u in range(QBLK):
                    dv = kk + av[u] * xk + bv[u] * yk + cv[u] * zk
                    accs[u] = jnp.minimum(accs[u], dv)
            return tuple(accs)

        accs = lax.fori_loop(0, NKV // UNROLL, k_body,
                             (inf,) * QBLK)
        lane0 = lax.iota(jnp.int32, L) == 0
        for u in range(QBLK):
            m = jnp.min(accs[u] + q2v[u])
            idx = jnp.full((L,), qbase + u, jnp.int32)
            plsc.store_scatter(res, [idx], jnp.full((L,), m), mask=lane0)
        return carry

    lax.fori_loop(0, _QLEN // QBLK, qblk_body, 0)

    pltpu.sync_copy(res, out_hbm.at[d, b, pl.ds(q0, _QLEN)])


def _make_sc_kernel():
    mesh = plsc.VectorSubcoreMesh(core_axis_name="c", subcore_axis_name="s")
    return pl.kernel(
        _sc_tile_body,
        out_type=jax.ShapeDtypeStruct((2, KSC, N), jnp.float32),
        mesh=mesh,
        scratch_types=[
            pltpu.VMEM((_QLEN,), jnp.float32),
            pltpu.VMEM((_QLEN,), jnp.float32),
            pltpu.VMEM((_QLEN,), jnp.float32),
            pltpu.VMEM((_QLEN,), jnp.float32),
            pltpu.VMEM((N,), jnp.float32),
            pltpu.VMEM((N,), jnp.float32),
            pltpu.VMEM((N,), jnp.float32),
            pltpu.VMEM((N,), jnp.float32),
            pltpu.VMEM((_QLEN,), jnp.float32),
        ],
        compiler_params=pltpu.CompilerParams(needs_layout_passes=False),
    )


def _tc_batch(xb, yt, d1_ref, d2_ref, b):
    n1 = jnp.sum(xb * xb, axis=1, keepdims=True)
    n2 = jnp.sum(yt * yt, axis=0, keepdims=True)
    hi = n2.astype(jnp.bfloat16)
    r1 = n2 - hi.astype(jnp.float32)
    mid = r1.astype(jnp.bfloat16)
    lo = (r1 - mid.astype(jnp.float32)).astype(jnp.bfloat16)
    a = (-2.0 * xb).astype(jnp.bfloat16)
    ones = jnp.ones((N, 3), jnp.bfloat16)
    lhs = jnp.concatenate([a, ones], axis=1)
    rhs = jnp.concatenate(
        [yt.astype(jnp.bfloat16), hi, mid, lo], axis=0)
    CW = N // 4
    NCH = N // CW
    d1acc = None
    prev = None

    def chunk_mins(c, t):
        nonlocal d1acc
        m = jnp.min(t, axis=1)
        d1acc = m if d1acc is None else jnp.minimum(d1acc, m)
        d2_ref[b, 0, c * CW:(c + 1) * CW] = jnp.min(t + n1, axis=0)

    for c in range(NCH):
        t = lax.dot_general(lhs, rhs[:, c * CW:(c + 1) * CW],
                            (((1,), (0,)), ((), ())),
                            preferred_element_type=jnp.float32)
        if prev is not None:
            chunk_mins(c - 1, prev)
        prev = t
    chunk_mins(NCH - 1, prev)
    d1_ref[b, 0, :] = d1acc + n1[:, 0]


def _tc_body(x1_ref, p2t_ref, d1_ref, d2_ref):
    nb = x1_ref.shape[0]

    def body(b, carry):
        _tc_batch(x1_ref[b], p2t_ref[b], d1_ref, d2_ref, b)
        return carry

    lax.fori_loop(0, nb, body, 0)


def _tc_pair(x1, x2t, nb_batches):
    return pl.pallas_call(
        _tc_body,
        out_shape=[
            jax.ShapeDtypeStruct((nb_batches, 1, N), jnp.float32),
            jax.ShapeDtypeStruct((nb_batches, 1, N), jnp.float32),
        ],
    )(x1, x2t)


def kernel(xyz1, xyz2):
    outs1 = []
    outs2 = []
    if KSC:
        pts = jnp.stack([
            jnp.moveaxis(xyz1[:KSC], -1, 0),
            jnp.moveaxis(xyz2[:KSC], -1, 0),
        ])
        sc_out = _make_sc_kernel()(pts)
        outs1.append(sc_out[0])
        outs2.append(sc_out[1])
    if KSC < B:
        ntc = B - KSC
        x1 = xyz1[KSC:]
        x2 = xyz2[KSC:]
        x2t = jnp.moveaxis(x2, -1, 1)
        d1, d2 = _tc_pair(x1, x2t, ntc)
        outs1.append(d1.reshape(ntc, N))
        outs2.append(d2.reshape(ntc, N))
    dist1 = outs1[0] if len(outs1) == 1 else jnp.concatenate(outs1, axis=0)
    dist2 = outs2[0] if len(outs2) == 1 else jnp.concatenate(outs2, axis=0)
    return dist1, dist2
